# baseline (speedup 1.0000x reference)
import jax
import jax.numpy as jnp
from jax import lax
from jax.experimental import pallas as pl
from jax.experimental.pallas import tpu as pltpu
from jax.experimental.pallas import tpu_sc as plsc

N = 320000
D = 128
NSEG = 10000
EPS = 1e-3

B1 = 3200
NTILES = 16
ROWS_PER_TILE = N // NTILES
STRIPE = 624

SBS = 20
SGRP = 8 * SBS
SGROUPS = ROWS_PER_TILE // SGRP

GBS = SBS
GW = 25
GIDR = 640
GGRP = 8 * GBS
GGROUPS = GIDR // 8


def _tc_pass1_body(feat_ref, xyz16_ref, wpre_ref, bpre_ref, w1_ref, w2_ref,
                   x1_ref, x2_ref, p16_ref, stats_ref):
    i = pl.program_id(0)
    feat = feat_ref[...]
    p16 = jnp.floor(xyz16_ref[...])
    t = jnp.dot(feat, wpre_ref[...], preferred_element_type=jnp.float32) + bpre_ref[...]
    pw1 = jnp.dot(p16, w1_ref[...], preferred_element_type=jnp.float32)
    pw2 = jnp.dot(p16, w2_ref[...], preferred_element_type=jnp.float32)
    x1 = pw1 * t
    x2 = pw2 * t
    x1_ref[...] = x1
    x2_ref[...] = x2
    p16_ref[...] = p16[:, 0:8]
    st = jnp.concatenate([
        jnp.sum(t, 0, keepdims=True),
        jnp.sum(t * t, 0, keepdims=True),
        jnp.sum(x2, 0, keepdims=True),
        jnp.sum(x2 * x2, 0, keepdims=True),
        jnp.sum(x2 * pw2, 0, keepdims=True),
        jnp.sum(pw2, 0, keepdims=True),
        jnp.sum(pw2 * pw2, 0, keepdims=True),
        jnp.zeros((1, D), jnp.float32),
    ], axis=0)

    @pl.when(i == 0)
    def _():
        stats_ref[...] = st

    @pl.when(i > 0)
    def _():
        stats_ref[...] += st


def _tc_pass1(feat_all, xyz16, wpre, bpre2, w1p, w2p):
    nblk = N // B1
    return pl.pallas_call(
        _tc_pass1_body,
        grid=(nblk,),
        in_specs=[
            pl.BlockSpec((B1, D), lambda i: (i, 0)),
            pl.BlockSpec((B1, 16), lambda i: (i, 0)),
            pl.BlockSpec((D, D), lambda i: (0, 0)),
            pl.BlockSpec((1, D), lambda i: (0, 0)),
            pl.BlockSpec((16, D), lambda i: (0, 0)),
            pl.BlockSpec((16, D), lambda i: (0, 0)),
        ],
        out_specs=[
            pl.BlockSpec((B1, D), lambda i: (i, 0)),
            pl.BlockSpec((B1, D), lambda i: (i, 0)),
            pl.BlockSpec((B1, 8), lambda i: (i, 0)),
            pl.BlockSpec((8, D), lambda i: (0, 0)),
        ],
        out_shape=[
            jax.ShapeDtypeStruct((N, D), jnp.float32),
            jax.ShapeDtypeStruct((N, D), jnp.float32),
            jax.ShapeDtypeStruct((N, 8), jnp.float32),
            jax.ShapeDtypeStruct((8, D), jnp.float32),
        ],
        compiler_params=pltpu.CompilerParams(
            dimension_semantics=("arbitrary",)),
    )(feat_all, xyz16, wpre, bpre2, w1p, w2p)


def _copy_striped(src_hbm, tab_sh, sid):
    @pl.when(sid < NTILES - 1)
    def _():
        off = pl.multiple_of(sid * STRIPE, 8)
        pltpu.sync_copy(src_hbm.at[pl.ds(off, STRIPE), :],
                        tab_sh.at[pl.ds(off, STRIPE), :])

    @pl.when(sid == NTILES - 1)
    def _():
        last = NSEG - (NTILES - 1) * STRIPE
        pltpu.sync_copy(src_hbm.at[pl.ds((NTILES - 1) * STRIPE, last), :],
                        tab_sh.at[pl.ds((NTILES - 1) * STRIPE, last), :])


def _copy_table_out(tab_sh, out_hbm, sid):
    @pl.when(sid < NTILES - 1)
    def _():
        off = pl.multiple_of(sid * STRIPE, 8)
        pltpu.sync_copy(tab_sh.at[pl.ds(off, STRIPE), :],
                        out_hbm.at[pl.ds(off, STRIPE), :])

    @pl.when(sid == NTILES - 1)
    def _():
        last = NSEG - (NTILES - 1) * STRIPE
        pltpu.sync_copy(tab_sh.at[pl.ds((NTILES - 1) * STRIPE, last), :],
                        out_hbm.at[pl.ds((NTILES - 1) * STRIPE, last), :])


def _scatter_pipeline(src_hbm, ids_hbm, tab_sh, sid, rows, idx, lsems, ssems,
                      g_lo=0, g_hi=SGROUPS):
    idrows_per_tile = ROWS_PER_TILE // SBS

    def issue_loads(g, b):
        base = pl.multiple_of(sid * ROWS_PER_TILE + g * SGRP, SGRP)
        idrow = pl.multiple_of(sid * idrows_per_tile + g * 8, 8)
        pltpu.async_copy(ids_hbm.at[pl.ds(idrow, 8), :], idx[b], lsems[b])
        pltpu.async_copy(src_hbm.at[pl.ds(base, SGRP), :], rows[b], lsems[b])

    issue_loads(g_lo, 0)
    issue_loads(g_lo + 1, 1)

    def body(g, b):
        pltpu.make_async_copy(ids_hbm.at[pl.ds(0, 8), :], idx[b], lsems[b]).wait()
        pltpu.make_async_copy(src_hbm.at[pl.ds(0, SGRP), :], rows[b], lsems[b]).wait()
        descs = []
        for j in range(8):
            descs.append(pltpu.async_copy(
                rows[b].at[pl.ds(j * SBS, SBS), :],
                tab_sh.at[idx[b].at[j]], ssems[b], add=True))
        for d in descs:
            d.wait()

        @pl.when(g + 2 < g_hi)
        def _():
            issue_loads(g + 2, b)

    def loop(g, _):
        @pl.when((g - g_lo) % 2 == 0)
        def _():
            body(g, 0)

        @pl.when((g - g_lo) % 2 == 1)
        def _():
            body(g, 1)
        return ()

    lax.fori_loop(g_lo, g_hi, loop, ())


def _sc_segsum_body(x1_hbm, x2_hbm, ids_hbm, z128_hbm,
                    t1_hbm, sa_hbm,
                    rows0, rows1, idx0, idx1, big_sh,
                    lsem0, lsem1, ssem0, ssem1):
    cid = lax.axis_index("c")
    sid = lax.axis_index("s")
    _copy_striped(z128_hbm, big_sh, sid)
    plsc.subcore_barrier()

    @pl.when(cid == 0)
    def _():
        _scatter_pipeline(x1_hbm, ids_hbm, big_sh, sid, (rows0, rows1),
                          (idx0, idx1), (lsem0, lsem1), (ssem0, ssem1))

    @pl.when(cid == 1)
    def _():
        _scatter_pipeline(x2_hbm, ids_hbm, big_sh, sid, (rows0, rows1),
                          (idx0, idx1), (lsem0, lsem1), (ssem0, ssem1))

    plsc.subcore_barrier()

    @pl.when(cid == 0)
    def _():
        _copy_table_out(big_sh, t1_hbm, sid)

    @pl.when(cid == 1)
    def _():
        _copy_table_out(big_sh, sa_hbm, sid)


def _sc_segsum(x1, x2, ids20, z128):
    mesh = plsc.VectorSubcoreMesh(core_axis_name="c", subcore_axis_name="s")
    f = pl.kernel(
        _sc_segsum_body,
        out_type=[
            jax.ShapeDtypeStruct((NSEG, D), jnp.float32),
            jax.ShapeDtypeStruct((NSEG, D), jnp.float32),
        ],
        mesh=mesh,
        scratch_types=[
            pltpu.VMEM((SGRP, D), jnp.float32),
            pltpu.VMEM((SGRP, D), jnp.float32),
            pltpu.VMEM((8, SBS), jnp.int32),
            pltpu.VMEM((8, SBS), jnp.int32),
            pltpu.VMEM_SHARED((NSEG, D), jnp.float32),
            pltpu.SemaphoreType.DMA,
            pltpu.SemaphoreType.DMA,
            pltpu.SemaphoreType.DMA,
            pltpu.SemaphoreType.DMA,
        ],
    )
    return f(x1, x2, ids20, z128)


def _sc_segsum_p_body(p8_hbm, ids_hbm, z8_hbm, sp_hbm,
                      rows0, rows1, idx0, idx1, sp_sh,
                      lsem0, lsem1, ssem0, ssem1):
    cid = lax.axis_index("c")
    sid = lax.axis_index("s")

    _copy_striped(z8_hbm, sp_sh, sid)
    plsc.subcore_barrier()
    g_lo = cid * (SGROUPS // 2 + 1)
    g_hi = jnp.where(cid == 0, SGROUPS // 2 + 1, SGROUPS)
    _scatter_pipeline(p8_hbm, ids_hbm, sp_sh, sid, (rows0, rows1),
                      (idx0, idx1), (lsem0, lsem1), (ssem0, ssem1),
                      g_lo=g_lo, g_hi=g_hi)
    plsc.subcore_barrier()
    _copy_table_out(sp_sh, sp_hbm.at[cid], sid)


def _sc_segsum_p(p8, ids20, z8):
    mesh = plsc.VectorSubcoreMesh(core_axis_name="c", subcore_axis_name="s")
    f = pl.kernel(
        _sc_segsum_p_body,
        out_type=jax.ShapeDtypeStruct((2, NSEG, 8), jnp.float32),
        mesh=mesh,
        scratch_types=[
            pltpu.VMEM((SGRP, 8), jnp.float32),
            pltpu.VMEM((SGRP, 8), jnp.float32),
            pltpu.VMEM((8, SBS), jnp.int32),
            pltpu.VMEM((8, SBS), jnp.int32),
            pltpu.VMEM_SHARED((NSEG, 8), jnp.float32),
            pltpu.SemaphoreType.DMA,
            pltpu.SemaphoreType.DMA,
            pltpu.SemaphoreType.DMA,
            pltpu.SemaphoreType.DMA,
        ],
    )
    return f(p8, ids20, z8)


def _sc_gather_body(gf_hbm, ids_hbm, g0_hbm,
                    rows0, rows1, idx0, idx1, tab_sh,
                    lsem0, lsem1, gsem0, gsem1, stsem0, stsem1):
    cid = lax.axis_index("c")
    sid = lax.axis_index("s")
    wid = sid * 2 + cid
    rows = (rows0, rows1)
    idx = (idx0, idx1)
    lsems = (lsem0, lsem1)
    gsems = (gsem0, gsem1)
    stsems = (stsem0, stsem1)

    _copy_striped(gf_hbm, tab_sh, sid)
    plsc.subcore_barrier()

    def issue_idx(g, b):
        idrow = pl.multiple_of(wid * GIDR + g * 8, 8)
        pltpu.async_copy(ids_hbm.at[pl.ds(idrow, 8), :], idx[b], lsems[b])

    @pl.when(wid < GW)
    def _():
        issue_idx(0, 0)
        issue_idx(1, 1)

        def body(g, b):
            pltpu.make_async_copy(ids_hbm.at[pl.ds(0, 8), :], idx[b],
                                  lsems[b]).wait()

            @pl.when(g >= 2)
            def _():
                pltpu.make_async_copy(rows[b], g0_hbm.at[pl.ds(0, GGRP), :],
                                      stsems[b]).wait()

            descs = []
            for j in range(8):
                descs.append(pltpu.async_copy(
                    tab_sh.at[idx[b].at[j]],
                    rows[b].at[pl.ds(j * GBS, GBS), :], gsems[b]))
            for d in descs:
                d.wait()
            base = pl.multiple_of(wid * GIDR * GBS + g * GGRP, 8)
            pltpu.async_copy(rows[b], g0_hbm.at[pl.ds(base, GGRP), :], stsems[b])

            @pl.when(g + 2 < GGROUPS)
            def _():
                issue_idx(g + 2, b)

        def loop(g, _):
            @pl.when(g % 2 == 0)
            def _():
                body(g, 0)

            @pl.when(g % 2 == 1)
            def _():
                body(g, 1)
            return ()

        lax.fori_loop(0, GGROUPS, loop, ())
        pltpu.make_async_copy(rows[0], g0_hbm.at[pl.ds(0, GGRP), :],
                              stsems[0]).wait()
        pltpu.make_async_copy(rows[1], g0_hbm.at[pl.ds(0, GGRP), :],
                              stsems[1]).wait()


def _sc_gather(gf, ids20):
    mesh = plsc.VectorSubcoreMesh(core_axis_name="c", subcore_axis_name="s")
    f = pl.kernel(
        _sc_gather_body,
        out_type=jax.ShapeDtypeStruct((N, D), jnp.float32),
        mesh=mesh,
        scratch_types=[
            pltpu.VMEM((GGRP, D), jnp.float32),
            pltpu.VMEM((GGRP, D), jnp.float32),
            pltpu.VMEM((8, GBS), jnp.int32),
            pltpu.VMEM((8, GBS), jnp.int32),
            pltpu.VMEM_SHARED((NSEG, D), jnp.float32),
            pltpu.SemaphoreType.DMA,
            pltpu.SemaphoreType.DMA,
            pltpu.SemaphoreType.DMA,
            pltpu.SemaphoreType.DMA,
            pltpu.SemaphoreType.DMA,
            pltpu.SemaphoreType.DMA,
        ],
    )
    return f(gf, ids20)


def _tc_stats_body(t1_ref, sa_ref, sp_ref, stats_ref, w1_ref, w2_ref, gb_ref,
                   gf_ref, k12_ref):
    t1 = t1_ref[...]
    sa = sa_ref[...]
    sp = sp_ref[0] + sp_ref[1]
    stats = stats_ref[...]
    g1 = gb_ref[0:1, :]
    be1 = gb_ref[1:2, :]
    g2 = gb_ref[2:3, :]
    be2 = gb_ref[3:4, :]
    fN = jnp.float32(N)

    mean1 = stats[0:1, :] / fN
    var1 = stats[1:2, :] / fN - mean1 * mean1
    a1 = g1 * lax.rsqrt(var1 + EPS)
    c1 = be1 - mean1 * a1

    p1 = jnp.dot(sp, w1_ref[...], preferred_element_type=jnp.float32)
    sp2 = jnp.dot(sp, w2_ref[...], preferred_element_type=jnp.float32)
    cnt = sp[:, 3:4]

    su = stats[2:3, :] - jnp.sum(cnt * t1, 0, keepdims=True)
    su2 = (stats[3:4, :] - 2.0 * jnp.sum(t1 * sa, 0, keepdims=True)
           + jnp.sum(cnt * t1 * t1, 0, keepdims=True))
    sv = stats[5:6, :] - jnp.sum(cnt * p1, 0, keepdims=True)
    sv2 = (stats[6:7, :] - 2.0 * jnp.sum(p1 * sp2, 0, keepdims=True)
           + jnp.sum(cnt * p1 * p1, 0, keepdims=True))
    suv = (stats[4:5, :] - jnp.sum(p1 * sa, 0, keepdims=True)
           - jnp.sum(t1 * sp2, 0, keepdims=True)
           + jnp.sum(cnt * t1 * p1, 0, keepdims=True))

    m2 = (a1 * su + c1 * sv) / fN
    eop2 = (a1 * a1 * su2 + 2.0 * a1 * c1 * suv + c1 * c1 * sv2) / fN
    var2 = eop2 - m2 * m2
    a2 = g2 * lax.rsqrt(var2 + EPS)
    c2 = be2 - m2 * a2
    k1 = a2 * a1
    k2 = a2 * c1
    gf_ref[...] = k1 * t1 + k2 * p1 - c2
    k12_ref[...] = jnp.concatenate([k1, k2], axis=0)


def _tc_stats(t1, sa, sp, stats, w1p8, w2p8, gb):
    return pl.pallas_call(
        _tc_stats_body,
        out_shape=[
            jax.ShapeDtypeStruct((NSEG, D), jnp.float32),
            jax.ShapeDtypeStruct((2, D), jnp.float32),
        ],
    )(t1, sa, sp, stats, w1p8, w2p8, gb)


def _tc_final_body(x2_ref, xyz16_ref, g0_ref, k12_ref, w2_ref, out_ref):
    p16 = jnp.floor(xyz16_ref[...])
    pw2 = jnp.dot(p16, w2_ref[...], preferred_element_type=jnp.float32)
    out = (k12_ref[0:1, :] * x2_ref[...] + k12_ref[1:2, :] * pw2
           - g0_ref[...])
    out_ref[...] = jnp.maximum(out, 0.0)


def _tc_final(x2, xyz16, g0, k12, w2p):
    nblk = N // B1
    return pl.pallas_call(
        _tc_final_body,
        grid=(nblk,),
        in_specs=[
            pl.BlockSpec((B1, D), lambda i: (i, 0)),
            pl.BlockSpec((B1, 4), lambda i: (i, 0)),
            pl.BlockSpec((B1, D), lambda i: (i, 0)),
            pl.BlockSpec((2, D), lambda i: (0, 0)),
            pl.BlockSpec((4, D), lambda i: (0, 0)),
        ],
        out_specs=pl.BlockSpec((B1, D), lambda i: (i, 0)),
        out_shape=jax.ShapeDtypeStruct((N, D), jnp.float32),
        compiler_params=pltpu.CompilerParams(
            dimension_semantics=("arbitrary",)),
    )(x2, xyz16, g0, k12, w2p)


def kernel(points_xyz, feat_all, unq_inv, W_pre, b_pre, gamma1, beta1,
           W_p1, b_p1, W_p2, b_p2, gamma2, beta2):
    ids32 = unq_inv.astype(jnp.int32)
    ids20 = ids32.reshape(N // SBS, SBS)
    xyz16 = jnp.concatenate(
        [points_xyz, jnp.ones((N, 1), jnp.float32),
         jnp.zeros((N, 12), jnp.float32)], axis=1)
    w1p = jnp.concatenate(
        [W_p1, b_p1[None, :], jnp.zeros((12, D), jnp.float32)], axis=0)
    w2p = jnp.concatenate(
        [W_p2, b_p2[None, :], jnp.zeros((12, D), jnp.float32)], axis=0)
    bpre2 = b_pre[None, :]
    gb = jnp.stack([gamma1, beta1, gamma2, beta2], axis=0)
    z128 = jnp.zeros((NSEG, D), jnp.float32)
    z8 = jnp.zeros((NSEG, 8), jnp.float32)

    x1, x2, p8, stats = _tc_pass1(feat_all, xyz16, W_pre, bpre2, w1p, w2p)
    t1, sa = _sc_segsum(x1, x2, ids20, z128)
    sp = _sc_segsum_p(p8, ids20, z8)
    gf, k12 = _tc_stats(t1, sa, sp, stats, w1p[0:8, :], w2p[0:8, :], gb)
    g0 = _sc_gather(gf, ids20)
    return _tc_final(x2, xyz16[:, 0:4], g0, k12, w2p[0:4, :])


# speedup vs baseline: 1.0537x; 1.0537x over previous
import jax
import jax.numpy as jnp
from jax import lax
from jax.experimental import pallas as pl
from jax.experimental.pallas import tpu as pltpu
from jax.experimental.pallas import tpu_sc as plsc

N = 320000
D = 128
NSEG = 10000
EPS = 1e-3

B1 = 3200
NTILES = 16
ROWS_PER_TILE = N // NTILES
STRIPE = 624

SBS = 20
SGRP = 8 * SBS
SGROUPS = ROWS_PER_TILE // SGRP

GBS = SBS
GW = 25
GIDR = 640
GGRP = 8 * GBS
GGROUPS = GIDR // 8


def _tc_pass1_body(feat_ref, xyz16_ref, wpre_ref, bpre_ref, w1_ref, w2_ref,
                   x1_ref, x2_ref, p16_ref, stats_ref):
    i = pl.program_id(0)
    feat = feat_ref[...]
    p16 = jnp.floor(xyz16_ref[...])
    t = jnp.dot(feat, wpre_ref[...], preferred_element_type=jnp.float32) + bpre_ref[...]
    pw1 = jnp.dot(p16, w1_ref[...], preferred_element_type=jnp.float32)
    pw2 = jnp.dot(p16, w2_ref[...], preferred_element_type=jnp.float32)
    x1 = pw1 * t
    x2 = pw2 * t
    x1_ref[...] = x1
    x2_ref[...] = x2
    p16_ref[...] = p16[:, 0:8]
    st = jnp.concatenate([
        jnp.sum(t, 0, keepdims=True),
        jnp.sum(t * t, 0, keepdims=True),
        jnp.sum(x2, 0, keepdims=True),
        jnp.sum(x2 * x2, 0, keepdims=True),
        jnp.sum(x2 * pw2, 0, keepdims=True),
        jnp.sum(pw2, 0, keepdims=True),
        jnp.sum(pw2 * pw2, 0, keepdims=True),
        jnp.zeros((1, D), jnp.float32),
    ], axis=0)

    @pl.when(i == 0)
    def _():
        stats_ref[...] = st

    @pl.when(i > 0)
    def _():
        stats_ref[...] += st


def _tc_pass1(feat_all, xyz16, wpre, bpre2, w1p, w2p):
    nblk = N // B1
    return pl.pallas_call(
        _tc_pass1_body,
        grid=(nblk,),
        in_specs=[
            pl.BlockSpec((B1, D), lambda i: (i, 0)),
            pl.BlockSpec((B1, 16), lambda i: (i, 0)),
            pl.BlockSpec((D, D), lambda i: (0, 0)),
            pl.BlockSpec((1, D), lambda i: (0, 0)),
            pl.BlockSpec((16, D), lambda i: (0, 0)),
            pl.BlockSpec((16, D), lambda i: (0, 0)),
        ],
        out_specs=[
            pl.BlockSpec((B1, D), lambda i: (i, 0)),
            pl.BlockSpec((B1, D), lambda i: (i, 0)),
            pl.BlockSpec((B1, 8), lambda i: (i, 0)),
            pl.BlockSpec((8, D), lambda i: (0, 0)),
        ],
        out_shape=[
            jax.ShapeDtypeStruct((N, D), jnp.float32),
            jax.ShapeDtypeStruct((N, D), jnp.float32),
            jax.ShapeDtypeStruct((N, 8), jnp.float32),
            jax.ShapeDtypeStruct((8, D), jnp.float32),
        ],
        compiler_params=pltpu.CompilerParams(
            dimension_semantics=("arbitrary",)),
    )(feat_all, xyz16, wpre, bpre2, w1p, w2p)


def _copy_striped(src_hbm, tab_sh, sid):
    @pl.when(sid < NTILES - 1)
    def _():
        off = pl.multiple_of(sid * STRIPE, 8)
        pltpu.sync_copy(src_hbm.at[pl.ds(off, STRIPE), :],
                        tab_sh.at[pl.ds(off, STRIPE), :])

    @pl.when(sid == NTILES - 1)
    def _():
        last = NSEG - (NTILES - 1) * STRIPE
        pltpu.sync_copy(src_hbm.at[pl.ds((NTILES - 1) * STRIPE, last), :],
                        tab_sh.at[pl.ds((NTILES - 1) * STRIPE, last), :])


def _copy_table_out(tab_sh, out_hbm, sid):
    @pl.when(sid < NTILES - 1)
    def _():
        off = pl.multiple_of(sid * STRIPE, 8)
        pltpu.sync_copy(tab_sh.at[pl.ds(off, STRIPE), :],
                        out_hbm.at[pl.ds(off, STRIPE), :])

    @pl.when(sid == NTILES - 1)
    def _():
        last = NSEG - (NTILES - 1) * STRIPE
        pltpu.sync_copy(tab_sh.at[pl.ds((NTILES - 1) * STRIPE, last), :],
                        out_hbm.at[pl.ds((NTILES - 1) * STRIPE, last), :])


def _scatter_pipeline(src_hbm, ids_hbm, tab_sh, sid, rows, idx, lsems, ssems,
                      g_lo=0, g_hi=SGROUPS):
    idrows_per_tile = ROWS_PER_TILE // SBS

    def issue_loads(g, b):
        base = pl.multiple_of(sid * ROWS_PER_TILE + g * SGRP, SGRP)
        idrow = pl.multiple_of(sid * idrows_per_tile + g * 8, 8)
        pltpu.async_copy(ids_hbm.at[pl.ds(idrow, 8), :], idx[b], lsems[b])
        pltpu.async_copy(src_hbm.at[pl.ds(base, SGRP), :], rows[b], lsems[b])

    issue_loads(g_lo, 0)
    issue_loads(g_lo + 1, 1)

    def body(g, b):
        pltpu.make_async_copy(ids_hbm.at[pl.ds(0, 8), :], idx[b], lsems[b]).wait()
        pltpu.make_async_copy(src_hbm.at[pl.ds(0, SGRP), :], rows[b], lsems[b]).wait()
        descs = []
        for j in range(8):
            descs.append(pltpu.async_copy(
                rows[b].at[pl.ds(j * SBS, SBS), :],
                tab_sh.at[idx[b].at[j]], ssems[b], add=True))
        for d in descs:
            d.wait()

        @pl.when(g + 2 < g_hi)
        def _():
            issue_loads(g + 2, b)

    def loop(g, _):
        @pl.when((g - g_lo) % 2 == 0)
        def _():
            body(g, 0)

        @pl.when((g - g_lo) % 2 == 1)
        def _():
            body(g, 1)
        return ()

    lax.fori_loop(g_lo, g_hi, loop, ())


def _sc_segsum_body(x1_hbm, x2_hbm, ids_hbm, z128_hbm,
                    t1_hbm, sa_hbm,
                    rows0, rows1, idx0, idx1, big_sh,
                    lsem0, lsem1, ssem0, ssem1):
    cid = lax.axis_index("c")
    sid = lax.axis_index("s")
    _copy_striped(z128_hbm, big_sh, sid)
    plsc.subcore_barrier()

    @pl.when(cid == 0)
    def _():
        _scatter_pipeline(x1_hbm, ids_hbm, big_sh, sid, (rows0, rows1),
                          (idx0, idx1), (lsem0, lsem1), (ssem0, ssem1))

    @pl.when(cid == 1)
    def _():
        _scatter_pipeline(x2_hbm, ids_hbm, big_sh, sid, (rows0, rows1),
                          (idx0, idx1), (lsem0, lsem1), (ssem0, ssem1))

    plsc.subcore_barrier()

    @pl.when(cid == 0)
    def _():
        _copy_table_out(big_sh, t1_hbm, sid)

    @pl.when(cid == 1)
    def _():
        _copy_table_out(big_sh, sa_hbm, sid)


def _sc_segsum(x1, x2, ids20, z128):
    mesh = plsc.VectorSubcoreMesh(core_axis_name="c", subcore_axis_name="s")
    f = pl.kernel(
        _sc_segsum_body,
        out_type=[
            jax.ShapeDtypeStruct((NSEG, D), jnp.float32),
            jax.ShapeDtypeStruct((NSEG, D), jnp.float32),
        ],
        mesh=mesh,
        scratch_types=[
            pltpu.VMEM((SGRP, D), jnp.float32),
            pltpu.VMEM((SGRP, D), jnp.float32),
            pltpu.VMEM((8, SBS), jnp.int32),
            pltpu.VMEM((8, SBS), jnp.int32),
            pltpu.VMEM_SHARED((NSEG, D), jnp.float32),
            pltpu.SemaphoreType.DMA,
            pltpu.SemaphoreType.DMA,
            pltpu.SemaphoreType.DMA,
            pltpu.SemaphoreType.DMA,
        ],
    )
    return f(x1, x2, ids20, z128)


def _sc_segsum_p_body(p8_hbm, ids_hbm, z8_hbm, sp_hbm,
                      rows0, rows1, idx0, idx1, sp_sh,
                      lsem0, lsem1, ssem0, ssem1):
    cid = lax.axis_index("c")
    sid = lax.axis_index("s")

    @pl.when(cid == 0)
    def _():
        _copy_striped(z8_hbm, sp_sh, sid)
        plsc.subcore_barrier()
        _scatter_pipeline(p8_hbm, ids_hbm, sp_sh, sid, (rows0, rows1),
                          (idx0, idx1), (lsem0, lsem1), (ssem0, ssem1))
        plsc.subcore_barrier()
        _copy_table_out(sp_sh, sp_hbm, sid)


def _sc_segsum_p(p8, ids20, z8):
    mesh = plsc.VectorSubcoreMesh(core_axis_name="c", subcore_axis_name="s")
    f = pl.kernel(
        _sc_segsum_p_body,
        out_type=jax.ShapeDtypeStruct((NSEG, 8), jnp.float32),
        mesh=mesh,
        scratch_types=[
            pltpu.VMEM((SGRP, 8), jnp.float32),
            pltpu.VMEM((SGRP, 8), jnp.float32),
            pltpu.VMEM((8, SBS), jnp.int32),
            pltpu.VMEM((8, SBS), jnp.int32),
            pltpu.VMEM_SHARED((NSEG, 8), jnp.float32),
            pltpu.SemaphoreType.DMA,
            pltpu.SemaphoreType.DMA,
            pltpu.SemaphoreType.DMA,
            pltpu.SemaphoreType.DMA,
        ],
    )
    return f(p8, ids20, z8)


def _sc_gather_body(gf_hbm, ids_hbm, g0_hbm,
                    rows0, rows1, idx0, idx1, tab_sh,
                    lsem0, lsem1, gsem0, gsem1, stsem0, stsem1):
    cid = lax.axis_index("c")
    sid = lax.axis_index("s")
    wid = sid * 2 + cid
    rows = (rows0, rows1)
    idx = (idx0, idx1)
    lsems = (lsem0, lsem1)
    gsems = (gsem0, gsem1)
    stsems = (stsem0, stsem1)

    _copy_striped(gf_hbm, tab_sh, sid)
    plsc.subcore_barrier()

    def issue_idx(g, b):
        idrow = pl.multiple_of(wid * GIDR + g * 8, 8)
        pltpu.async_copy(ids_hbm.at[pl.ds(idrow, 8), :], idx[b], lsems[b])

    @pl.when(wid < GW)
    def _():
        issue_idx(0, 0)
        issue_idx(1, 1)

        def body(g, b):
            pltpu.make_async_copy(ids_hbm.at[pl.ds(0, 8), :], idx[b],
                                  lsems[b]).wait()

            @pl.when(g >= 2)
            def _():
                pltpu.make_async_copy(rows[b], g0_hbm.at[pl.ds(0, GGRP), :],
                                      stsems[b]).wait()

            descs = []
            for j in range(8):
                descs.append(pltpu.async_copy(
                    tab_sh.at[idx[b].at[j]],
                    rows[b].at[pl.ds(j * GBS, GBS), :], gsems[b]))
            for d in descs:
                d.wait()
            base = pl.multiple_of(wid * GIDR * GBS + g * GGRP, 8)
            pltpu.async_copy(rows[b], g0_hbm.at[pl.ds(base, GGRP), :], stsems[b])

            @pl.when(g + 2 < GGROUPS)
            def _():
                issue_idx(g + 2, b)

        def loop(g, _):
            @pl.when(g % 2 == 0)
            def _():
                body(g, 0)

            @pl.when(g % 2 == 1)
            def _():
                body(g, 1)
            return ()

        lax.fori_loop(0, GGROUPS, loop, ())
        pltpu.make_async_copy(rows[0], g0_hbm.at[pl.ds(0, GGRP), :],
                              stsems[0]).wait()
        pltpu.make_async_copy(rows[1], g0_hbm.at[pl.ds(0, GGRP), :],
                              stsems[1]).wait()


def _sc_gather(gf, ids20):
    mesh = plsc.VectorSubcoreMesh(core_axis_name="c", subcore_axis_name="s")
    f = pl.kernel(
        _sc_gather_body,
        out_type=jax.ShapeDtypeStruct((N, D), jnp.float32),
        mesh=mesh,
        scratch_types=[
            pltpu.VMEM((GGRP, D), jnp.float32),
            pltpu.VMEM((GGRP, D), jnp.float32),
            pltpu.VMEM((8, GBS), jnp.int32),
            pltpu.VMEM((8, GBS), jnp.int32),
            pltpu.VMEM_SHARED((NSEG, D), jnp.float32),
            pltpu.SemaphoreType.DMA,
            pltpu.SemaphoreType.DMA,
            pltpu.SemaphoreType.DMA,
            pltpu.SemaphoreType.DMA,
            pltpu.SemaphoreType.DMA,
            pltpu.SemaphoreType.DMA,
        ],
    )
    return f(gf, ids20)


def _tc_stats_body(t1_ref, sa_ref, sp_ref, stats_ref, w1_ref, w2_ref, gb_ref,
                   gf_ref, k12_ref):
    t1 = t1_ref[...]
    sa = sa_ref[...]
    sp = sp_ref[...]
    stats = stats_ref[...]
    g1 = gb_ref[0:1, :]
    be1 = gb_ref[1:2, :]
    g2 = gb_ref[2:3, :]
    be2 = gb_ref[3:4, :]
    fN = jnp.float32(N)

    mean1 = stats[0:1, :] / fN
    var1 = stats[1:2, :] / fN - mean1 * mean1
    a1 = g1 * lax.rsqrt(var1 + EPS)
    c1 = be1 - mean1 * a1

    p1 = jnp.dot(sp, w1_ref[...], preferred_element_type=jnp.float32)
    sp2 = jnp.dot(sp, w2_ref[...], preferred_element_type=jnp.float32)
    cnt = sp[:, 3:4]

    su = stats[2:3, :] - jnp.sum(cnt * t1, 0, keepdims=True)
    su2 = (stats[3:4, :] - 2.0 * jnp.sum(t1 * sa, 0, keepdims=True)
           + jnp.sum(cnt * t1 * t1, 0, keepdims=True))
    sv = stats[5:6, :] - jnp.sum(cnt * p1, 0, keepdims=True)
    sv2 = (stats[6:7, :] - 2.0 * jnp.sum(p1 * sp2, 0, keepdims=True)
           + jnp.sum(cnt * p1 * p1, 0, keepdims=True))
    suv = (stats[4:5, :] - jnp.sum(p1 * sa, 0, keepdims=True)
           - jnp.sum(t1 * sp2, 0, keepdims=True)
           + jnp.sum(cnt * t1 * p1, 0, keepdims=True))

    m2 = (a1 * su + c1 * sv) / fN
    eop2 = (a1 * a1 * su2 + 2.0 * a1 * c1 * suv + c1 * c1 * sv2) / fN
    var2 = eop2 - m2 * m2
    a2 = g2 * lax.rsqrt(var2 + EPS)
    c2 = be2 - m2 * a2
    k1 = a2 * a1
    k2 = a2 * c1
    gf_ref[...] = k1 * t1 + k2 * p1 - c2
    k12_ref[...] = jnp.concatenate([k1, k2], axis=0)


def _tc_stats(t1, sa, sp, stats, w1p8, w2p8, gb):
    return pl.pallas_call(
        _tc_stats_body,
        out_shape=[
            jax.ShapeDtypeStruct((NSEG, D), jnp.float32),
            jax.ShapeDtypeStruct((2, D), jnp.float32),
        ],
    )(t1, sa, sp, stats, w1p8, w2p8, gb)


def _tc_final_body(x2_ref, xyz16_ref, g0_ref, k12_ref, w2_ref, out_ref):
    p16 = jnp.floor(xyz16_ref[...])
    pw2 = jnp.dot(p16, w2_ref[...], preferred_element_type=jnp.float32)
    out = (k12_ref[0:1, :] * x2_ref[...] + k12_ref[1:2, :] * pw2
           - g0_ref[...])
    out_ref[...] = jnp.maximum(out, 0.0)


def _tc_final(x2, xyz16, g0, k12, w2p):
    nblk = N // B1
    return pl.pallas_call(
        _tc_final_body,
        grid=(nblk,),
        in_specs=[
            pl.BlockSpec((B1, D), lambda i: (i, 0)),
            pl.BlockSpec((B1, 16), lambda i: (i, 0)),
            pl.BlockSpec((B1, D), lambda i: (i, 0)),
            pl.BlockSpec((2, D), lambda i: (0, 0)),
            pl.BlockSpec((16, D), lambda i: (0, 0)),
        ],
        out_specs=pl.BlockSpec((B1, D), lambda i: (i, 0)),
        out_shape=jax.ShapeDtypeStruct((N, D), jnp.float32),
        compiler_params=pltpu.CompilerParams(
            dimension_semantics=("arbitrary",)),
    )(x2, xyz16, g0, k12, w2p)


def kernel(points_xyz, feat_all, unq_inv, W_pre, b_pre, gamma1, beta1,
           W_p1, b_p1, W_p2, b_p2, gamma2, beta2):
    ids32 = unq_inv.astype(jnp.int32)
    ids20 = ids32.reshape(N // SBS, SBS)
    xyz16 = jnp.concatenate(
        [points_xyz, jnp.ones((N, 1), jnp.float32),
         jnp.zeros((N, 12), jnp.float32)], axis=1)
    w1p = jnp.concatenate(
        [W_p1, b_p1[None, :], jnp.zeros((12, D), jnp.float32)], axis=0)
    w2p = jnp.concatenate(
        [W_p2, b_p2[None, :], jnp.zeros((12, D), jnp.float32)], axis=0)
    bpre2 = b_pre[None, :]
    gb = jnp.stack([gamma1, beta1, gamma2, beta2], axis=0)
    z128 = jnp.zeros((NSEG, D), jnp.float32)
    z8 = jnp.zeros((NSEG, 8), jnp.float32)

    x1, x2, p8, stats = _tc_pass1(feat_all, xyz16, W_pre, bpre2, w1p, w2p)
    t1, sa = _sc_segsum(x1, x2, ids20, z128)
    sp = _sc_segsum_p(p8, ids20, z8)
    gf, k12 = _tc_stats(t1, sa, sp, stats, w1p[0:8, :], w2p[0:8, :], gb)
    g0 = _sc_gather(gf, ids20)
    return _tc_final(x2, xyz16, g0, k12, w2p)


# speedup vs baseline: 1.0642x; 1.0100x over previous
import jax
import jax.numpy as jnp
from jax import lax
from jax.experimental import pallas as pl
from jax.experimental.pallas import tpu as pltpu
from jax.experimental.pallas import tpu_sc as plsc

N = 320000
D = 128
NSEG = 10000
EPS = 1e-3

B1 = 6400
NTILES = 16
ROWS_PER_TILE = N // NTILES
STRIPE = 624

SBS = 20
SGRP = 8 * SBS
SGROUPS = ROWS_PER_TILE // SGRP

GBS = SBS
GW = 25
GIDR = 640
GGRP = 8 * GBS
GGROUPS = GIDR // 8


def _tc_pass1_body(feat_ref, xyz16_ref, wpre_ref, bpre_ref, w1_ref, w2_ref,
                   x1_ref, x2_ref, p16_ref, stats_ref):
    i = pl.program_id(0)
    feat = feat_ref[...]
    p16 = jnp.floor(xyz16_ref[...])
    t = jnp.dot(feat, wpre_ref[...], preferred_element_type=jnp.float32) + bpre_ref[...]
    pw1 = jnp.dot(p16, w1_ref[...], preferred_element_type=jnp.float32)
    pw2 = jnp.dot(p16, w2_ref[...], preferred_element_type=jnp.float32)
    x1 = pw1 * t
    x2 = pw2 * t
    x1_ref[...] = x1
    x2_ref[...] = x2
    p16_ref[...] = p16[:, 0:8]
    st = jnp.concatenate([
        jnp.sum(t, 0, keepdims=True),
        jnp.sum(t * t, 0, keepdims=True),
        jnp.sum(x2, 0, keepdims=True),
        jnp.sum(x2 * x2, 0, keepdims=True),
        jnp.sum(x2 * pw2, 0, keepdims=True),
        jnp.sum(pw2, 0, keepdims=True),
        jnp.sum(pw2 * pw2, 0, keepdims=True),
        jnp.zeros((1, D), jnp.float32),
    ], axis=0)

    @pl.when(i == 0)
    def _():
        stats_ref[...] = st

    @pl.when(i > 0)
    def _():
        stats_ref[...] += st


def _tc_pass1(feat_all, xyz16, wpre, bpre2, w1p, w2p):
    nblk = N // B1
    return pl.pallas_call(
        _tc_pass1_body,
        grid=(nblk,),
        in_specs=[
            pl.BlockSpec((B1, D), lambda i: (i, 0)),
            pl.BlockSpec((B1, 16), lambda i: (i, 0)),
            pl.BlockSpec((D, D), lambda i: (0, 0)),
            pl.BlockSpec((1, D), lambda i: (0, 0)),
            pl.BlockSpec((16, D), lambda i: (0, 0)),
            pl.BlockSpec((16, D), lambda i: (0, 0)),
        ],
        out_specs=[
            pl.BlockSpec((B1, D), lambda i: (i, 0)),
            pl.BlockSpec((B1, D), lambda i: (i, 0)),
            pl.BlockSpec((B1, 8), lambda i: (i, 0)),
            pl.BlockSpec((8, D), lambda i: (0, 0)),
        ],
        out_shape=[
            jax.ShapeDtypeStruct((N, D), jnp.float32),
            jax.ShapeDtypeStruct((N, D), jnp.float32),
            jax.ShapeDtypeStruct((N, 8), jnp.float32),
            jax.ShapeDtypeStruct((8, D), jnp.float32),
        ],
        compiler_params=pltpu.CompilerParams(
            dimension_semantics=("arbitrary",)),
    )(feat_all, xyz16, wpre, bpre2, w1p, w2p)


def _copy_striped(src_hbm, tab_sh, sid):
    @pl.when(sid < NTILES - 1)
    def _():
        off = pl.multiple_of(sid * STRIPE, 8)
        pltpu.sync_copy(src_hbm.at[pl.ds(off, STRIPE), :],
                        tab_sh.at[pl.ds(off, STRIPE), :])

    @pl.when(sid == NTILES - 1)
    def _():
        last = NSEG - (NTILES - 1) * STRIPE
        pltpu.sync_copy(src_hbm.at[pl.ds((NTILES - 1) * STRIPE, last), :],
                        tab_sh.at[pl.ds((NTILES - 1) * STRIPE, last), :])


def _copy_table_out(tab_sh, out_hbm, sid):
    @pl.when(sid < NTILES - 1)
    def _():
        off = pl.multiple_of(sid * STRIPE, 8)
        pltpu.sync_copy(tab_sh.at[pl.ds(off, STRIPE), :],
                        out_hbm.at[pl.ds(off, STRIPE), :])

    @pl.when(sid == NTILES - 1)
    def _():
        last = NSEG - (NTILES - 1) * STRIPE
        pltpu.sync_copy(tab_sh.at[pl.ds((NTILES - 1) * STRIPE, last), :],
                        out_hbm.at[pl.ds((NTILES - 1) * STRIPE, last), :])


def _scatter_pipeline(src_hbm, ids_hbm, tab_sh, sid, rows, idx, lsems, ssems,
                      g_lo=0, g_hi=SGROUPS):
    idrows_per_tile = ROWS_PER_TILE // SBS

    def issue_loads(g, b):
        base = pl.multiple_of(sid * ROWS_PER_TILE + g * SGRP, SGRP)
        idrow = pl.multiple_of(sid * idrows_per_tile + g * 8, 8)
        pltpu.async_copy(ids_hbm.at[pl.ds(idrow, 8), :], idx[b], lsems[b])
        pltpu.async_copy(src_hbm.at[pl.ds(base, SGRP), :], rows[b], lsems[b])

    issue_loads(g_lo, 0)
    issue_loads(g_lo + 1, 1)

    def body(g, b):
        pltpu.make_async_copy(ids_hbm.at[pl.ds(0, 8), :], idx[b], lsems[b]).wait()
        pltpu.make_async_copy(src_hbm.at[pl.ds(0, SGRP), :], rows[b], lsems[b]).wait()
        descs = []
        for j in range(8):
            descs.append(pltpu.async_copy(
                rows[b].at[pl.ds(j * SBS, SBS), :],
                tab_sh.at[idx[b].at[j]], ssems[b], add=True))
        for d in descs:
            d.wait()

        @pl.when(g + 2 < g_hi)
        def _():
            issue_loads(g + 2, b)

    def loop(g, _):
        @pl.when((g - g_lo) % 2 == 0)
        def _():
            body(g, 0)

        @pl.when((g - g_lo) % 2 == 1)
        def _():
            body(g, 1)
        return ()

    lax.fori_loop(g_lo, g_hi, loop, ())


def _sc_segsum_body(x1_hbm, x2_hbm, ids_hbm, z128_hbm,
                    t1_hbm, sa_hbm,
                    rows0, rows1, idx0, idx1, big_sh,
                    lsem0, lsem1, ssem0, ssem1):
    cid = lax.axis_index("c")
    sid = lax.axis_index("s")
    _copy_striped(z128_hbm, big_sh, sid)
    plsc.subcore_barrier()

    @pl.when(cid == 0)
    def _():
        _scatter_pipeline(x1_hbm, ids_hbm, big_sh, sid, (rows0, rows1),
                          (idx0, idx1), (lsem0, lsem1), (ssem0, ssem1))

    @pl.when(cid == 1)
    def _():
        _scatter_pipeline(x2_hbm, ids_hbm, big_sh, sid, (rows0, rows1),
                          (idx0, idx1), (lsem0, lsem1), (ssem0, ssem1))

    plsc.subcore_barrier()

    @pl.when(cid == 0)
    def _():
        _copy_table_out(big_sh, t1_hbm, sid)

    @pl.when(cid == 1)
    def _():
        _copy_table_out(big_sh, sa_hbm, sid)


def _sc_segsum(x1, x2, ids20, z128):
    mesh = plsc.VectorSubcoreMesh(core_axis_name="c", subcore_axis_name="s")
    f = pl.kernel(
        _sc_segsum_body,
        out_type=[
            jax.ShapeDtypeStruct((NSEG, D), jnp.float32),
            jax.ShapeDtypeStruct((NSEG, D), jnp.float32),
        ],
        mesh=mesh,
        scratch_types=[
            pltpu.VMEM((SGRP, D), jnp.float32),
            pltpu.VMEM((SGRP, D), jnp.float32),
            pltpu.VMEM((8, SBS), jnp.int32),
            pltpu.VMEM((8, SBS), jnp.int32),
            pltpu.VMEM_SHARED((NSEG, D), jnp.float32),
            pltpu.SemaphoreType.DMA,
            pltpu.SemaphoreType.DMA,
            pltpu.SemaphoreType.DMA,
            pltpu.SemaphoreType.DMA,
        ],
    )
    return f(x1, x2, ids20, z128)


def _sc_segsum_p_body(p8_hbm, ids_hbm, z8_hbm, sp_hbm,
                      rows0, rows1, idx0, idx1, sp_sh,
                      lsem0, lsem1, ssem0, ssem1):
    cid = lax.axis_index("c")
    sid = lax.axis_index("s")

    @pl.when(cid == 0)
    def _():
        _copy_striped(z8_hbm, sp_sh, sid)
        plsc.subcore_barrier()
        _scatter_pipeline(p8_hbm, ids_hbm, sp_sh, sid, (rows0, rows1),
                          (idx0, idx1), (lsem0, lsem1), (ssem0, ssem1))
        plsc.subcore_barrier()
        _copy_table_out(sp_sh, sp_hbm, sid)


def _sc_segsum_p(p8, ids20, z8):
    mesh = plsc.VectorSubcoreMesh(core_axis_name="c", subcore_axis_name="s")
    f = pl.kernel(
        _sc_segsum_p_body,
        out_type=jax.ShapeDtypeStruct((NSEG, 8), jnp.float32),
        mesh=mesh,
        scratch_types=[
            pltpu.VMEM((SGRP, 8), jnp.float32),
            pltpu.VMEM((SGRP, 8), jnp.float32),
            pltpu.VMEM((8, SBS), jnp.int32),
            pltpu.VMEM((8, SBS), jnp.int32),
            pltpu.VMEM_SHARED((NSEG, 8), jnp.float32),
            pltpu.SemaphoreType.DMA,
            pltpu.SemaphoreType.DMA,
            pltpu.SemaphoreType.DMA,
            pltpu.SemaphoreType.DMA,
        ],
    )
    return f(p8, ids20, z8)


def _sc_gather_body(gf_hbm, ids_hbm, g0_hbm,
                    rows0, rows1, idx0, idx1, tab_sh,
                    lsem0, lsem1, gsem0, gsem1, stsem0, stsem1):
    cid = lax.axis_index("c")
    sid = lax.axis_index("s")
    wid = sid * 2 + cid
    rows = (rows0, rows1)
    idx = (idx0, idx1)
    lsems = (lsem0, lsem1)
    gsems = (gsem0, gsem1)
    stsems = (stsem0, stsem1)

    _copy_striped(gf_hbm, tab_sh, sid)
    plsc.subcore_barrier()

    def issue_idx(g, b):
        idrow = pl.multiple_of(wid * GIDR + g * 8, 8)
        pltpu.async_copy(ids_hbm.at[pl.ds(idrow, 8), :], idx[b], lsems[b])

    @pl.when(wid < GW)
    def _():
        issue_idx(0, 0)
        issue_idx(1, 1)

        def body(g, b):
            pltpu.make_async_copy(ids_hbm.at[pl.ds(0, 8), :], idx[b],
                                  lsems[b]).wait()

            @pl.when(g >= 2)
            def _():
                pltpu.make_async_copy(rows[b], g0_hbm.at[pl.ds(0, GGRP), :],
                                      stsems[b]).wait()

            descs = []
            for j in range(8):
                descs.append(pltpu.async_copy(
                    tab_sh.at[idx[b].at[j]],
                    rows[b].at[pl.ds(j * GBS, GBS), :], gsems[b]))
            for d in descs:
                d.wait()
            base = pl.multiple_of(wid * GIDR * GBS + g * GGRP, 8)
            pltpu.async_copy(rows[b], g0_hbm.at[pl.ds(base, GGRP), :], stsems[b])

            @pl.when(g + 2 < GGROUPS)
            def _():
                issue_idx(g + 2, b)

        def loop(g, _):
            @pl.when(g % 2 == 0)
            def _():
                body(g, 0)

            @pl.when(g % 2 == 1)
            def _():
                body(g, 1)
            return ()

        lax.fori_loop(0, GGROUPS, loop, ())
        pltpu.make_async_copy(rows[0], g0_hbm.at[pl.ds(0, GGRP), :],
                              stsems[0]).wait()
        pltpu.make_async_copy(rows[1], g0_hbm.at[pl.ds(0, GGRP), :],
                              stsems[1]).wait()


def _sc_gather(gf, ids20):
    mesh = plsc.VectorSubcoreMesh(core_axis_name="c", subcore_axis_name="s")
    f = pl.kernel(
        _sc_gather_body,
        out_type=jax.ShapeDtypeStruct((N, D), jnp.float32),
        mesh=mesh,
        scratch_types=[
            pltpu.VMEM((GGRP, D), jnp.float32),
            pltpu.VMEM((GGRP, D), jnp.float32),
            pltpu.VMEM((8, GBS), jnp.int32),
            pltpu.VMEM((8, GBS), jnp.int32),
            pltpu.VMEM_SHARED((NSEG, D), jnp.float32),
            pltpu.SemaphoreType.DMA,
            pltpu.SemaphoreType.DMA,
            pltpu.SemaphoreType.DMA,
            pltpu.SemaphoreType.DMA,
            pltpu.SemaphoreType.DMA,
            pltpu.SemaphoreType.DMA,
        ],
    )
    return f(gf, ids20)


def _tc_stats_body(t1_ref, sa_ref, sp_ref, stats_ref, w1_ref, w2_ref, gb_ref,
                   gf_ref, k12_ref):
    t1 = t1_ref[...]
    sa = sa_ref[...]
    sp = sp_ref[...]
    stats = stats_ref[...]
    g1 = gb_ref[0:1, :]
    be1 = gb_ref[1:2, :]
    g2 = gb_ref[2:3, :]
    be2 = gb_ref[3:4, :]
    fN = jnp.float32(N)

    mean1 = stats[0:1, :] / fN
    var1 = stats[1:2, :] / fN - mean1 * mean1
    a1 = g1 * lax.rsqrt(var1 + EPS)
    c1 = be1 - mean1 * a1

    p1 = jnp.dot(sp, w1_ref[...], preferred_element_type=jnp.float32)
    sp2 = jnp.dot(sp, w2_ref[...], preferred_element_type=jnp.float32)
    cnt = sp[:, 3:4]

    su = stats[2:3, :] - jnp.sum(cnt * t1, 0, keepdims=True)
    su2 = (stats[3:4, :] - 2.0 * jnp.sum(t1 * sa, 0, keepdims=True)
           + jnp.sum(cnt * t1 * t1, 0, keepdims=True))
    sv = stats[5:6, :] - jnp.sum(cnt * p1, 0, keepdims=True)
    sv2 = (stats[6:7, :] - 2.0 * jnp.sum(p1 * sp2, 0, keepdims=True)
           + jnp.sum(cnt * p1 * p1, 0, keepdims=True))
    suv = (stats[4:5, :] - jnp.sum(p1 * sa, 0, keepdims=True)
           - jnp.sum(t1 * sp2, 0, keepdims=True)
           + jnp.sum(cnt * t1 * p1, 0, keepdims=True))

    m2 = (a1 * su + c1 * sv) / fN
    eop2 = (a1 * a1 * su2 + 2.0 * a1 * c1 * suv + c1 * c1 * sv2) / fN
    var2 = eop2 - m2 * m2
    a2 = g2 * lax.rsqrt(var2 + EPS)
    c2 = be2 - m2 * a2
    k1 = a2 * a1
    k2 = a2 * c1
    gf_ref[...] = k1 * t1 + k2 * p1 - c2
    k12_ref[...] = jnp.concatenate([k1, k2], axis=0)


def _tc_stats(t1, sa, sp, stats, w1p8, w2p8, gb):
    return pl.pallas_call(
        _tc_stats_body,
        out_shape=[
            jax.ShapeDtypeStruct((NSEG, D), jnp.float32),
            jax.ShapeDtypeStruct((2, D), jnp.float32),
        ],
    )(t1, sa, sp, stats, w1p8, w2p8, gb)


def _tc_final_body(x2_ref, xyz16_ref, g0_ref, k12_ref, w2_ref, out_ref):
    p16 = jnp.floor(xyz16_ref[...])
    pw2 = jnp.dot(p16, w2_ref[...], preferred_element_type=jnp.float32)
    out = (k12_ref[0:1, :] * x2_ref[...] + k12_ref[1:2, :] * pw2
           - g0_ref[...])
    out_ref[...] = jnp.maximum(out, 0.0)


def _tc_final(x2, xyz16, g0, k12, w2p):
    nblk = N // B1
    return pl.pallas_call(
        _tc_final_body,
        grid=(nblk,),
        in_specs=[
            pl.BlockSpec((B1, D), lambda i: (i, 0)),
            pl.BlockSpec((B1, 16), lambda i: (i, 0)),
            pl.BlockSpec((B1, D), lambda i: (i, 0)),
            pl.BlockSpec((2, D), lambda i: (0, 0)),
            pl.BlockSpec((16, D), lambda i: (0, 0)),
        ],
        out_specs=pl.BlockSpec((B1, D), lambda i: (i, 0)),
        out_shape=jax.ShapeDtypeStruct((N, D), jnp.float32),
        compiler_params=pltpu.CompilerParams(
            dimension_semantics=("arbitrary",)),
    )(x2, xyz16, g0, k12, w2p)


def kernel(points_xyz, feat_all, unq_inv, W_pre, b_pre, gamma1, beta1,
           W_p1, b_p1, W_p2, b_p2, gamma2, beta2):
    ids32 = unq_inv.astype(jnp.int32)
    ids20 = ids32.reshape(N // SBS, SBS)
    xyz16 = jnp.concatenate(
        [points_xyz, jnp.ones((N, 1), jnp.float32),
         jnp.zeros((N, 12), jnp.float32)], axis=1)
    w1p = jnp.concatenate(
        [W_p1, b_p1[None, :], jnp.zeros((12, D), jnp.float32)], axis=0)
    w2p = jnp.concatenate(
        [W_p2, b_p2[None, :], jnp.zeros((12, D), jnp.float32)], axis=0)
    bpre2 = b_pre[None, :]
    gb = jnp.stack([gamma1, beta1, gamma2, beta2], axis=0)
    z128 = jnp.zeros((NSEG, D), jnp.float32)
    z8 = jnp.zeros((NSEG, 8), jnp.float32)

    x1, x2, p8, stats = _tc_pass1(feat_all, xyz16, W_pre, bpre2, w1p, w2p)
    t1, sa = _sc_segsum(x1, x2, ids20, z128)
    sp = _sc_segsum_p(p8, ids20, z8)
    gf, k12 = _tc_stats(t1, sa, sp, stats, w1p[0:8, :], w2p[0:8, :], gb)
    g0 = _sc_gather(gf, ids20)
    return _tc_final(x2, xyz16, g0, k12, w2p)
